# Initial kernel scaffold; baseline (speedup 1.0000x reference)
import jax
import jax.numpy as jnp
from jax.experimental import pallas as pl


def kernel(h, x, edges, coord_diff, distances, distance_org, W1, b1, W2, b2, W3):
    raise NotImplementedError("write your pallas kernel here")


# speedup vs baseline: 2.3069x; 2.3069x over previous
import functools
import jax
import jax.numpy as jnp
from jax import lax
from jax.experimental import pallas as pl
from jax.experimental.pallas import tpu as pltpu, tpu_sc as plsc

HIDDEN = 128
N_NODES = 10000
N_EDGES = 320000
SCALE = (12.0 / 6.0) / 100.0

NC = 2
NS = 16
NW = NC * NS
CHUNK = 128
NCHUNKS = N_EDGES // CHUNK
ITERS = (NCHUNKS + NW - 1) // NW
NP_PAD = 10240
ZROWS = NP_PAD // NS
TW = 8


def _nodeproj_body(h_ref, w1a_ref, w1b_ref, b1_ref, a_ref, b_ref):
    hb = h_ref[...]
    a_ref[...] = jnp.dot(hb, w1a_ref[...], preferred_element_type=jnp.float32) + b1_ref[...]
    b_ref[...] = jnp.dot(hb, w1b_ref[...], preferred_element_type=jnp.float32)


def _node_proj(h, w1a, w1b, b1r):
    blk = 2000
    grid = N_NODES // blk
    return pl.pallas_call(
        _nodeproj_body,
        grid=(grid,),
        in_specs=[
            pl.BlockSpec((blk, HIDDEN), lambda i: (i, 0)),
            pl.BlockSpec((HIDDEN, HIDDEN), lambda i: (0, 0)),
            pl.BlockSpec((HIDDEN, HIDDEN), lambda i: (0, 0)),
            pl.BlockSpec((1, HIDDEN), lambda i: (0, 0)),
        ],
        out_specs=[
            pl.BlockSpec((blk, HIDDEN), lambda i: (i, 0)),
            pl.BlockSpec((blk, HIDDEN), lambda i: (i, 0)),
        ],
        out_shape=[
            jax.ShapeDtypeStruct((N_NODES, HIDDEN), jnp.float32),
            jax.ShapeDtypeStruct((N_NODES, HIDDEN), jnp.float32),
        ],
    )(h, w1a, w1b, b1r)


def _gather_body(a_hbm, b_hbm, row_hbm, col_hbm, g1_hbm, g2_hbm,
                 idx1_v, idx2_v, r1_v, r2_v, sem1, sem2):
    w = lax.axis_index("s") * NC + lax.axis_index("c")

    def step(j, carry):
        c = w + NW * j

        @pl.when(c < NCHUNKS)
        def _():
            base = c * CHUNK
            pltpu.sync_copy(row_hbm.at[pl.ds(base, CHUNK)], idx1_v)
            pltpu.sync_copy(col_hbm.at[pl.ds(base, CHUNK)], idx2_v)
            cp1 = pltpu.async_copy(a_hbm.at[idx1_v], r1_v, sem1)
            cp2 = pltpu.async_copy(b_hbm.at[idx2_v], r2_v, sem2)
            cp1.wait()
            cp2.wait()
            pltpu.sync_copy(r1_v, g1_hbm.at[pl.ds(base, CHUNK)])
            pltpu.sync_copy(r2_v, g2_hbm.at[pl.ds(base, CHUNK)])

        return carry

    lax.fori_loop(0, ITERS, step, 0)


def _sc_gather(a, b, row, col):
    mesh = plsc.VectorSubcoreMesh(core_axis_name="c", subcore_axis_name="s", num_cores=NC, num_subcores=NS)
    kern = pl.kernel(
        _gather_body,
        out_type=[
            jax.ShapeDtypeStruct((N_EDGES, HIDDEN), jnp.float32),
            jax.ShapeDtypeStruct((N_EDGES, HIDDEN), jnp.float32),
        ],
        mesh=mesh,
        scratch_types=[
            pltpu.VMEM((CHUNK,), jnp.int32),
            pltpu.VMEM((CHUNK,), jnp.int32),
            pltpu.VMEM((CHUNK, HIDDEN), jnp.float32),
            pltpu.VMEM((CHUNK, HIDDEN), jnp.float32),
            pltpu.SemaphoreType.DMA,
            pltpu.SemaphoreType.DMA,
        ],
    )
    return kern(a, b, row, col)


def _edgemlp_body(g1_ref, g2_ref, d_ref, do_ref, cd_ref,
                  w1d_ref, w1e_ref, w2_ref, b2_ref, w3_ref, out_ref):
    s = (g1_ref[...] + g2_ref[...]
         + d_ref[...] * w1d_ref[...]
         + do_ref[...] * w1e_ref[...])
    t1 = s * (1.0 / (1.0 + jnp.exp(-s)))
    t2p = jnp.dot(t1, w2_ref[...], preferred_element_type=jnp.float32) + b2_ref[...]
    t2 = t2p * (1.0 / (1.0 + jnp.exp(-t2p)))
    t3 = jnp.dot(t2, w3_ref[...], preferred_element_type=jnp.float32)
    out_ref[...] = cd_ref[...] * (jnp.tanh(t3) * SCALE)


def _edge_mlp(g1, g2, d, do_, cd8, w1d, w1e, W2, b2r, W3):
    blk = 1280
    grid = N_EDGES // blk
    return pl.pallas_call(
        _edgemlp_body,
        grid=(grid,),
        in_specs=[
            pl.BlockSpec((blk, HIDDEN), lambda i: (i, 0)),
            pl.BlockSpec((blk, HIDDEN), lambda i: (i, 0)),
            pl.BlockSpec((blk, 1), lambda i: (i, 0)),
            pl.BlockSpec((blk, 1), lambda i: (i, 0)),
            pl.BlockSpec((blk, TW), lambda i: (i, 0)),
            pl.BlockSpec((1, HIDDEN), lambda i: (0, 0)),
            pl.BlockSpec((1, HIDDEN), lambda i: (0, 0)),
            pl.BlockSpec((HIDDEN, HIDDEN), lambda i: (0, 0)),
            pl.BlockSpec((1, HIDDEN), lambda i: (0, 0)),
            pl.BlockSpec((HIDDEN, 1), lambda i: (0, 0)),
        ],
        out_specs=pl.BlockSpec((blk, TW), lambda i: (i, 0)),
        out_shape=jax.ShapeDtypeStruct((N_EDGES, TW), jnp.float32),
    )(g1, g2, d, do_, cd8, w1d, w1e, W2, b2r, W3)


def _scatter_body(trans_hbm, row_hbm, zero_hbm, out_hbm, idx_v, t_v, acc_v):
    cid = lax.axis_index("c")
    sid = lax.axis_index("s")
    w = sid * NC + cid

    pltpu.sync_copy(zero_hbm, acc_v)

    def step(j, carry):
        c = w + NW * j

        @pl.when(c < NCHUNKS)
        def _():
            base = c * CHUNK
            pltpu.sync_copy(row_hbm.at[pl.ds(base, CHUNK)], idx_v)
            pltpu.sync_copy(trans_hbm.at[pl.ds(base * TW, CHUNK * TW)], t_v)
            lane = lax.iota(jnp.int32, 16)
            for k in range(CHUNK // 16):
                e16 = (lane + (k * 16)) * TW
                row16 = idx_v[pl.ds(k * 16, 16)] * TW
                for comp in range(3):
                    vals = plsc.load_gather(t_v, [e16 + comp])
                    plsc.addupdate_scatter(acc_v, [row16 + comp], vals)

        return carry

    lax.fori_loop(0, ITERS, step, 0)

    pltpu.sync_copy(acc_v, out_hbm.at[pl.ds(w * NP_PAD * TW, NP_PAD * TW)])


def _sc_scatter(trans_flat, row, zeros_flat):
    mesh = plsc.VectorSubcoreMesh(core_axis_name="c", subcore_axis_name="s", num_cores=NC, num_subcores=NS)
    kern = pl.kernel(
        _scatter_body,
        out_type=jax.ShapeDtypeStruct((NW * NP_PAD * TW,), jnp.float32),
        mesh=mesh,
        compiler_params=pltpu.CompilerParams(needs_layout_passes=False),
        scratch_types=[
            pltpu.VMEM((CHUNK,), jnp.int32),
            pltpu.VMEM((CHUNK * TW,), jnp.float32),
            pltpu.VMEM((NP_PAD * TW,), jnp.float32),
        ],
    )
    return kern(trans_flat, row, zeros_flat)


def _reduce_body(p_ref, x8_ref, out_ref):
    out_ref[...] = x8_ref[...] + jnp.sum(p_ref[...], axis=0)


def _tc_reduce(partials, x8):
    blk = 1024
    grid = NP_PAD // blk
    return pl.pallas_call(
        _reduce_body,
        grid=(grid,),
        in_specs=[
            pl.BlockSpec((NW, blk, TW), lambda i: (0, i, 0)),
            pl.BlockSpec((blk, TW), lambda i: (i, 0)),
        ],
        out_specs=pl.BlockSpec((blk, TW), lambda i: (i, 0)),
        out_shape=jax.ShapeDtypeStruct((NP_PAD, TW), jnp.float32),
    )(partials, x8)


@jax.jit
def kernel(h, x, edges, coord_diff, distances, distance_org, W1, b1, W2, b2, W3):
    row = edges[0].astype(jnp.int32)
    col = edges[1].astype(jnp.int32)

    w1a = W1[:HIDDEN]
    w1b = W1[HIDDEN:2 * HIDDEN]
    w1d = W1[2 * HIDDEN].reshape(1, HIDDEN)
    w1e = W1[2 * HIDDEN + 1].reshape(1, HIDDEN)
    b1r = b1.reshape(1, HIDDEN)
    b2r = b2.reshape(1, HIDDEN)

    a, b = _node_proj(h, w1a, w1b, b1r)
    g1, g2 = _sc_gather(a, b, row, col)

    cd8 = jnp.pad(coord_diff, ((0, 0), (0, TW - 3)))
    trans = _edge_mlp(g1, g2, distances, distance_org, cd8, w1d, w1e, W2, b2r, W3)

    zeros_flat = jnp.zeros((NP_PAD * TW,), jnp.float32)
    partials = _sc_scatter(trans.reshape(-1), row, zeros_flat).reshape(NW, NP_PAD, TW)

    x8 = jnp.pad(x, ((0, NP_PAD - N_NODES), (0, TW - 3)))
    out = _tc_reduce(partials, x8)
    return out[:N_NODES, :3]
